# Initial kernel scaffold; baseline (speedup 1.0000x reference)
import jax
import jax.numpy as jnp
from jax.experimental import pallas as pl


def kernel(x, embeddings):
    raise NotImplementedError("write your pallas kernel here")


# speedup vs baseline: 1.8154x; 1.8154x over previous
import functools

import jax
import jax.numpy as jnp
from jax import lax
from jax.experimental import pallas as pl
from jax.experimental.pallas import tpu as pltpu
from jax.experimental.pallas import tpu_sc as plsc

_B = 36864
_D = 64
_NE = 1024
_BR = 512
_GRID = _B // _BR

_NW = 32
_BPW = _B // _NW
_CH = 128
_NCH = _BPW // _CH


def _tc_body(x_ref, emb_ref, idx_ref, losssum_ref):
    i = pl.program_id(0)
    xb = x_ref[...]
    emb = emb_ref[...]
    sim = jnp.dot(xb, emb)
    rowsq = jnp.sum(xb * xb, axis=1, keepdims=True)
    embsq = jnp.sum(emb * emb, axis=0, keepdims=True)
    d = (rowsq + embsq) - 2.0 * sim
    idx_ref[...] = jnp.argmin(d, axis=1).astype(jnp.int32).reshape(_BR, 1)
    s = jnp.sum(jnp.min(d, axis=1))

    @pl.when(i == 0)
    def _():
        losssum_ref[0, 0] = s

    @pl.when(i > 0)
    def _():
        losssum_ref[0, 0] = losssum_ref[0, 0] + s


_tc_call = pl.pallas_call(
    _tc_body,
    grid=(_GRID,),
    in_specs=[
        pl.BlockSpec((_BR, _D), lambda i: (i, 0)),
        pl.BlockSpec((_D, _NE), lambda i: (0, 0)),
    ],
    out_specs=[
        pl.BlockSpec((_BR, 1), lambda i: (i, 0)),
        pl.BlockSpec(memory_space=pltpu.SMEM, block_shape=(1, 1),
                     index_map=lambda i: (0, 0)),
    ],
    out_shape=[
        jax.ShapeDtypeStruct((_B, 1), jnp.int32),
        jax.ShapeDtypeStruct((1, 1), jnp.float32),
    ],
    compiler_params=pltpu.CompilerParams(
        dimension_semantics=("arbitrary",),
    ),
)


@functools.cache
def _make_sc_gather():
    mesh = plsc.VectorSubcoreMesh(core_axis_name="c", subcore_axis_name="s")

    @functools.partial(
        pl.kernel,
        mesh=mesh,
        out_type=jax.ShapeDtypeStruct((_B, _D), jnp.float32),
        scratch_types=[
            pltpu.VMEM((_BPW,), jnp.int32),
            pltpu.VMEM((_BPW, _D), jnp.float32),
            pltpu.SemaphoreType.DMA,
        ],
        compiler_params=pltpu.CompilerParams(use_tc_tiling_on_sc=False),
    )
    def _sc_gather(table_hbm, idx_hbm, out_hbm, idx_v, rows_v, sem):
        wid = lax.axis_index("s") * 2 + lax.axis_index("c")
        base = wid * _BPW
        pltpu.sync_copy(idx_hbm.at[pl.ds(base, _BPW)], idx_v)
        copies = []
        for c in range(_NCH):
            copies.append(
                pltpu.async_copy(
                    table_hbm.at[idx_v.at[pl.ds(c * _CH, _CH)]],
                    rows_v.at[pl.ds(c * _CH, _CH)],
                    sem,
                ))
        for cp in copies:
            cp.wait()
        pltpu.sync_copy(rows_v, out_hbm.at[pl.ds(base, _BPW)])

    return _sc_gather


def kernel(x, embeddings):
    flat = jnp.reshape(x, (-1, _D))
    idx2, losssum = _tc_call(flat, embeddings)
    quant_flat = _make_sc_gather()(embeddings.T, jnp.reshape(idx2, (-1,)))
    quantized = jnp.reshape(quant_flat, x.shape)
    loss = 1.25 * (losssum[0, 0] / jnp.float32(_B * _D))
    return quantized, loss


# speedup vs baseline: 1.8303x; 1.0082x over previous
import functools

import jax
import jax.numpy as jnp
from jax import lax
from jax.experimental import pallas as pl
from jax.experimental.pallas import tpu as pltpu
from jax.experimental.pallas import tpu_sc as plsc

_B = 36864
_D = 64
_NE = 1024
_BR = 1024
_GRID = _B // _BR

_NW = 32
_BPW = _B // _NW
_CH = 128
_NCH = _BPW // _CH


def _tc_body(x_ref, emb_ref, idx_ref, embt_ref, losssum_ref):
    i = pl.program_id(0)
    xb = x_ref[...]
    emb = emb_ref[...]
    sim = jnp.dot(xb, emb)
    rowsq = jnp.sum(xb * xb, axis=1, keepdims=True)
    embsq = jnp.sum(emb * emb, axis=0, keepdims=True)
    d = (rowsq + embsq) - 2.0 * sim
    idx = jnp.argmin(d, axis=1).astype(jnp.int32)
    idx_ref[...] = idx.reshape(_BR // 128, 128)
    s = jnp.sum(jnp.min(d, axis=1))

    @pl.when(i == 0)
    def _():
        embt_ref[...] = emb.T
        losssum_ref[0, 0] = s

    @pl.when(i > 0)
    def _():
        losssum_ref[0, 0] = losssum_ref[0, 0] + s


_tc_call = pl.pallas_call(
    _tc_body,
    grid=(_GRID,),
    in_specs=[
        pl.BlockSpec((_BR, _D), lambda i: (i, 0)),
        pl.BlockSpec((_D, _NE), lambda i: (0, 0)),
    ],
    out_specs=[
        pl.BlockSpec((_BR // 128, 128), lambda i: (i, 0)),
        pl.BlockSpec((_NE, _D), lambda i: (0, 0)),
        pl.BlockSpec(memory_space=pltpu.SMEM, block_shape=(1, 1),
                     index_map=lambda i: (0, 0)),
    ],
    out_shape=[
        jax.ShapeDtypeStruct((_B // 128, 128), jnp.int32),
        jax.ShapeDtypeStruct((_NE, _D), jnp.float32),
        jax.ShapeDtypeStruct((1, 1), jnp.float32),
    ],
    compiler_params=pltpu.CompilerParams(
        dimension_semantics=("arbitrary",),
    ),
)


@functools.cache
def _make_sc_gather():
    mesh = plsc.VectorSubcoreMesh(core_axis_name="c", subcore_axis_name="s")

    @functools.partial(
        pl.kernel,
        mesh=mesh,
        out_type=jax.ShapeDtypeStruct((_B, _D), jnp.float32),
        scratch_types=[
            pltpu.VMEM((_BPW,), jnp.int32),
            pltpu.VMEM((_BPW, _D), jnp.float32),
            pltpu.SemaphoreType.DMA,
        ],
        compiler_params=pltpu.CompilerParams(use_tc_tiling_on_sc=False),
    )
    def _sc_gather(table_hbm, idx_hbm, out_hbm, idx_v, rows_v, sem):
        wid = lax.axis_index("s") * 2 + lax.axis_index("c")
        base = wid * _BPW
        pltpu.sync_copy(idx_hbm.at[pl.ds(base, _BPW)], idx_v)
        copies = []
        for c in range(_NCH):
            copies.append(
                pltpu.async_copy(
                    table_hbm.at[idx_v.at[pl.ds(c * _CH, _CH)]],
                    rows_v.at[pl.ds(c * _CH, _CH)],
                    sem,
                ))
        for cp in copies:
            cp.wait()
        pltpu.sync_copy(rows_v, out_hbm.at[pl.ds(base, _BPW)])

    return _sc_gather


def kernel(x, embeddings):
    flat = jnp.reshape(x, (-1, _D))
    idx2, embt, losssum = _tc_call(flat, embeddings)
    quant_flat = _make_sc_gather()(embt, jnp.reshape(idx2, (-1,)))
    quantized = jnp.reshape(quant_flat, x.shape)
    loss = 1.25 * (losssum[0, 0] / jnp.float32(_B * _D))
    return quantized, loss


# speedup vs baseline: 1.8357x; 1.0029x over previous
import functools

import jax
import jax.numpy as jnp
from jax import lax
from jax.experimental import pallas as pl
from jax.experimental.pallas import tpu as pltpu
from jax.experimental.pallas import tpu_sc as plsc

_B = 36864
_D = 64
_NE = 1024
_BR = 1024
_GRID = _B // _BR

_NW = 32
_BPW = _B // _NW
_CH = 128
_NCH = _BPW // _CH


_HB = _BR // 2


def _tc_body(x_ref, emb_ref, idx_ref, losssum_ref):
    i = pl.program_id(0)
    emb = emb_ref[...]
    embsq = jnp.sum(emb * emb, axis=0, keepdims=True)

    total = jnp.float32(0.0)
    for h in range(2):
        xb = x_ref[pl.ds(h * _HB, _HB), :]
        sim = jnp.dot(xb, emb)
        rowsq = jnp.sum(xb * xb, axis=1, keepdims=True)
        d = (rowsq + embsq) - 2.0 * sim
        idx = jnp.argmin(d, axis=1).astype(jnp.int32)
        idx_ref[pl.ds(h * (_HB // 128), _HB // 128), :] = (
            idx.reshape(_HB // 128, 128))
        total = total + jnp.sum(jnp.min(d, axis=1))

    @pl.when(i == 0)
    def _():
        losssum_ref[0, 0] = total

    @pl.when(i > 0)
    def _():
        losssum_ref[0, 0] = losssum_ref[0, 0] + total


_tc_call = pl.pallas_call(
    _tc_body,
    grid=(_GRID,),
    in_specs=[
        pl.BlockSpec((_BR, _D), lambda i: (i, 0)),
        pl.BlockSpec((_D, _NE), lambda i: (0, 0)),
    ],
    out_specs=[
        pl.BlockSpec((_BR // 128, 128), lambda i: (i, 0)),
        pl.BlockSpec(memory_space=pltpu.SMEM, block_shape=(1, 1),
                     index_map=lambda i: (0, 0)),
    ],
    out_shape=[
        jax.ShapeDtypeStruct((_B // 128, 128), jnp.int32),
        jax.ShapeDtypeStruct((1, 1), jnp.float32),
    ],
    compiler_params=pltpu.CompilerParams(
        dimension_semantics=("arbitrary",),
    ),
)


def _embt_body(emb_ref, embt_ref):
    embt_ref[...] = emb_ref[...].T


_embt_call = pl.pallas_call(
    _embt_body,
    out_shape=jax.ShapeDtypeStruct((_NE, _D), jnp.float32),
)


@functools.cache
def _make_sc_gather():
    mesh = plsc.VectorSubcoreMesh(core_axis_name="c", subcore_axis_name="s")

    @functools.partial(
        pl.kernel,
        mesh=mesh,
        out_type=jax.ShapeDtypeStruct((_B, _D), jnp.float32),
        scratch_types=[
            pltpu.VMEM((_BPW,), jnp.int32),
            pltpu.VMEM((_BPW, _D), jnp.float32),
            pltpu.SemaphoreType.DMA,
        ],
        compiler_params=pltpu.CompilerParams(use_tc_tiling_on_sc=False),
    )
    def _sc_gather(table_hbm, idx_hbm, out_hbm, idx_v, rows_v, sem):
        wid = lax.axis_index("s") * 2 + lax.axis_index("c")
        base = wid * _BPW
        pltpu.sync_copy(idx_hbm.at[pl.ds(base, _BPW)], idx_v)
        copies = []
        for c in range(_NCH):
            copies.append(
                pltpu.async_copy(
                    table_hbm.at[idx_v.at[pl.ds(c * _CH, _CH)]],
                    rows_v.at[pl.ds(c * _CH, _CH)],
                    sem,
                ))
        for cp in copies:
            cp.wait()
        pltpu.sync_copy(rows_v, out_hbm.at[pl.ds(base, _BPW)])

    return _sc_gather


def kernel(x, embeddings):
    flat = jnp.reshape(x, (-1, _D))
    embt = _embt_call(embeddings)
    idx2, losssum = _tc_call(flat, embeddings)
    quant_flat = _make_sc_gather()(embt, jnp.reshape(idx2, (-1,)))
    quantized = jnp.reshape(quant_flat, x.shape)
    loss = 1.25 * (losssum[0, 0] / jnp.float32(_B * _D))
    return quantized, loss
